# baseline (speedup 1.0000x reference)
import functools

import jax
import jax.numpy as jnp
from jax import lax
from jax.experimental import pallas as pl
from jax.experimental.pallas import tpu as pltpu
from jax.experimental.pallas import tpu_sc as plsc

VOCAB = 1000000
DIM = 64
BATCH = 4096
SEQ = 200

NC = 2
NS = 16
NW = NC * NS
L = 16

CHUNK = 128
NBUF = 2
TBUF = 2


def _gather_body(idx2_hbm, off_hbm, table_hbm, out_hbm,
                 idx2_v, off_v, rows_v, tbuf, *sems):
    gsems = sems[:NBUF]
    osems = sems[NBUF:]
    wid = lax.axis_index("s") * NC + lax.axis_index("c")
    base = wid * CHUNK
    pltpu.sync_copy(idx2_hbm.at[:, pl.ds(base, CHUNK)], idx2_v)
    pltpu.sync_copy(off_hbm.at[:, pl.ds(base, CHUNK)], off_v)

    lane = jax.lax.iota(jnp.int32, L)
    perms = [(lane + k) & (L - 1) for k in range(L)]

    for b in range(NBUF):
        pltpu.async_copy(table_hbm.at[idx2_v.at[b]], rows_v.at[b], gsems[b])

    @pl.loop(0, SEQ, step=NBUF)
    def _(g):
        for b in range(NBUF):
            s = g + b
            t = b
            pltpu.make_async_copy(
                table_hbm.at[pl.ds(0, CHUNK)], rows_v.at[b], gsems[b]
            ).wait()
            @pl.when(s >= TBUF)
            def _():
                pltpu.make_async_copy(
                    tbuf.at[t], out_hbm.at[0, :, pl.ds(base, CHUNK)], osems[t]
                ).wait()

            @plsc.parallel_loop(0, CHUNK // L)
            def _(jb):
                jids = lane + jb * L
                offs = off_v[s, pl.ds(jb * L, L)]
                for db in range(DIM // L):
                    for k in range(L):
                        dcol = perms[k] + (db * L)
                        v = plsc.load_gather(rows_v.at[b], [jids, offs + dcol])
                        plsc.store_scatter(tbuf.at[t], [dcol, jids], v)

            pltpu.async_copy(tbuf.at[t], out_hbm.at[s, :, pl.ds(base, CHUNK)],
                             osems[t])

            @pl.when(s + NBUF < SEQ)
            def _():
                pltpu.async_copy(
                    table_hbm.at[idx2_v.at[s + NBUF]], rows_v.at[b], gsems[b]
                )

    for t in range(TBUF):
        pltpu.make_async_copy(
            tbuf.at[t], out_hbm.at[0, :, pl.ds(base, CHUNK)], osems[t]
        ).wait()


@jax.jit
def _embed(idx2, off, table2):
    mesh = plsc.VectorSubcoreMesh(
        core_axis_name="c", subcore_axis_name="s",
        num_cores=NC, num_subcores=NS,
    )
    run = pl.kernel(
        _gather_body,
        out_type=jax.ShapeDtypeStruct((SEQ, DIM, BATCH), jnp.float32),
        mesh=mesh,
        scratch_types=[
            pltpu.VMEM((SEQ, CHUNK), jnp.int32),
            pltpu.VMEM((SEQ, CHUNK), jnp.int32),
            pltpu.VMEM((NBUF, CHUNK, 2 * DIM), jnp.float32),
            pltpu.VMEM((TBUF, DIM, CHUNK), jnp.float32),
        ] + [pltpu.SemaphoreType.DMA] * (NBUF + TBUF),
        compiler_params=pltpu.CompilerParams(
            use_tc_tiling_on_sc=True, needs_layout_passes=False),
    )
    return run(idx2, off, table2)


def kernel(input, table):
    inpT = input.T
    idx2 = inpT >> 1
    off = (inpT & 1) << 6
    table2 = table.reshape(VOCAB // 2, 2 * DIM)
    out = _embed(idx2, off, table2)
    return out.transpose(2, 0, 1)
